# baseline (device time: 16116 ns/iter reference)
import jax
import jax.numpy as jnp
from jax import lax
from jax.experimental import pallas as pl
from jax.experimental.pallas import tpu as pltpu

N_DEV = 4
B, SQ, D = 2, 128, 512
HQ, HKV, DH = 8, 2, 64
GRP = HQ // HKV
SKV_LOC = 128
KV_W = HKV * DH
SCALE = 0.125


def kernel(x, Wq, Wo, K_ext, V_ext):
    K2 = K_ext.reshape(B, SKV_LOC, KV_W)
    V2 = V_ext.reshape(B, SKV_LOC, KV_W)

    def body(x_hbm, wq_hbm, wo_hbm, k_ref, v_ref, out_ref,
             kv_send, kv_full, x_v, wq_v, wo_v,
             send_sems, recv_sems, copy_sems):
        my = lax.axis_index("i")

        kv_send[:, :, 0:KV_W] = k_ref[...].astype(jnp.bfloat16)
        kv_send[:, :, KV_W:2 * KV_W] = v_ref[...].astype(jnp.bfloat16)

        barrier_sem = pltpu.get_barrier_semaphore()
        for d_rel in range(1, N_DEV):
            peer = (my + d_rel) % N_DEV
            pl.semaphore_signal(
                barrier_sem, inc=1,
                device_id=(peer,), device_id_type=pl.DeviceIdType.MESH,
            )
        pl.semaphore_wait(barrier_sem, N_DEV - 1)

        sends = []
        for j, d_rel in enumerate((2, 1, 3)):
            peer = (my + d_rel) % N_DEV
            rd = pltpu.make_async_remote_copy(
                src_ref=kv_send,
                dst_ref=kv_full.at[my],
                send_sem=send_sems.at[j],
                recv_sem=recv_sems.at[my],
                device_id=(peer,),
                device_id_type=pl.DeviceIdType.MESH,
            )
            rd.start()
            sends.append(rd)

        cps = [
            pltpu.make_async_copy(x_hbm, x_v, copy_sems.at[0]),
            pltpu.make_async_copy(wq_hbm, wq_v, copy_sems.at[1]),
            pltpu.make_async_copy(wo_hbm, wo_v, copy_sems.at[2]),
        ]
        for cp in cps:
            cp.start()
        cps[0].wait()
        cps[1].wait()

        Qs = {}
        for b in range(B):
            q = jnp.dot(x_v[b], wq_v[...],
                        preferred_element_type=jnp.float32)
            qb = q.astype(jnp.bfloat16)
            for g in range(HKV):
                Qs[b, g] = jnp.concatenate(
                    [qb[:, (g * GRP + j) * DH:(g * GRP + j + 1) * DH]
                     for j in range(GRP)], axis=0)

        st = {}

        def process(b, g, Kc, Vc):
            s = lax.dot_general(
                Qs[b, g], Kc, (((1,), (1,)), ((), ())),
                preferred_element_type=jnp.float32) * SCALE
            p = jnp.exp(s)
            l = jnp.sum(p, axis=-1, keepdims=True)
            pv = jnp.dot(p.astype(jnp.bfloat16), Vc,
                         preferred_element_type=jnp.float32)
            if (b, g) not in st:
                st[b, g] = (l, pv)
            else:
                l0, acc = st[b, g]
                st[b, g] = (l0 + l, acc + pv)

        for b in range(B):
            for g in range(HKV):
                process(b, g,
                        kv_send[b, :, g * DH:(g + 1) * DH],
                        kv_send[b, :, KV_W + g * DH:KV_W + (g + 1) * DH])

        cps[2].wait()
        wo_b = wo_v[...].astype(jnp.bfloat16)

        for d_rel in (1, 3, 2):
            origin = (my + d_rel) % N_DEV
            wr = pltpu.make_async_remote_copy(
                src_ref=kv_full.at[0],
                dst_ref=kv_full.at[origin],
                send_sem=send_sems.at[0],
                recv_sem=recv_sems.at[origin],
                device_id=(my,),
                device_id_type=pl.DeviceIdType.MESH,
            )
            wr.wait_recv()
            for b in range(B):
                for g in range(HKV):
                    process(b, g,
                            kv_full[origin, b, :, g * DH:(g + 1) * DH],
                            kv_full[origin, b, :,
                                    KV_W + g * DH:KV_W + (g + 1) * DH])

        for b in range(B):
            blocks = []
            for h in range(HQ):
                g, j = divmod(h, GRP)
                l, acc = st[b, g]
                o = (acc / l).astype(jnp.bfloat16)
                blocks.append(o[j * SQ:(j + 1) * SQ, :])
            att = jnp.concatenate(blocks, axis=1)
            out_ref[b, :, :] = jnp.dot(att, wo_b,
                                       preferred_element_type=jnp.float32)

        for rd in sends:
            rd.wait_send()

    return pl.pallas_call(
        body,
        out_shape=jax.ShapeDtypeStruct((B, SQ, D), jnp.float32),
        in_specs=[
            pl.BlockSpec(memory_space=pltpu.MemorySpace.HBM),
            pl.BlockSpec(memory_space=pltpu.MemorySpace.HBM),
            pl.BlockSpec(memory_space=pltpu.MemorySpace.HBM),
            pl.BlockSpec(memory_space=pltpu.VMEM),
            pl.BlockSpec(memory_space=pltpu.VMEM),
        ],
        out_specs=pl.BlockSpec(memory_space=pltpu.VMEM),
        scratch_shapes=[
            pltpu.VMEM((B, SKV_LOC, 2 * KV_W), jnp.bfloat16),
            pltpu.VMEM((N_DEV, B, SKV_LOC, 2 * KV_W), jnp.bfloat16),
            pltpu.VMEM((B, SQ, D), jnp.float32),
            pltpu.VMEM((D, D), jnp.float32),
            pltpu.VMEM((D, D), jnp.float32),
            pltpu.SemaphoreType.DMA((N_DEV - 1,)),
            pltpu.SemaphoreType.DMA((N_DEV,)),
            pltpu.SemaphoreType.DMA((3,)),
        ],
        compiler_params=pltpu.CompilerParams(collective_id=0),
    )(x, Wq, Wo, K2, V2)


# device time: 13640 ns/iter; 1.1815x vs baseline; 1.1815x over previous
import jax
import jax.numpy as jnp
from jax import lax
from jax.experimental import pallas as pl
from jax.experimental.pallas import tpu as pltpu

N_DEV = 4
B, SQ, D = 2, 128, 512
HQ, HKV, DH = 8, 2, 64
GRP = HQ // HKV
SKV_LOC = 128
KV_W = HKV * DH
SCALE = 0.125


def kernel(x, Wq, Wo, K_ext, V_ext):
    kv = jnp.concatenate(
        [K_ext.reshape(B, SKV_LOC, KV_W), V_ext.reshape(B, SKV_LOC, KV_W)],
        axis=-1).astype(jnp.bfloat16)
    xb = x.astype(jnp.bfloat16)
    wqb = Wq.astype(jnp.bfloat16)
    wob = Wo.astype(jnp.bfloat16)

    def body(x_ref, wq_ref, wo_ref, kv_ref, out_ref,
             kv_full, send_sems, recv_sems):
        my = lax.axis_index("i")

        barrier_sem = pltpu.get_barrier_semaphore()
        for d_rel in range(1, N_DEV):
            peer = (my + d_rel) % N_DEV
            pl.semaphore_signal(
                barrier_sem, inc=1,
                device_id=(peer,), device_id_type=pl.DeviceIdType.MESH,
            )
        pl.semaphore_wait(barrier_sem, N_DEV - 1)

        sends = []
        for j, d_rel in enumerate((2, 1, 3)):
            peer = (my + d_rel) % N_DEV
            rd = pltpu.make_async_remote_copy(
                src_ref=kv_ref,
                dst_ref=kv_full.at[my],
                send_sem=send_sems.at[j],
                recv_sem=recv_sems.at[my],
                device_id=(peer,),
                device_id_type=pl.DeviceIdType.MESH,
            )
            rd.start()
            sends.append(rd)

        Qs = {}
        for b in range(B):
            q = jnp.dot(x_ref[b], wq_ref[...],
                        preferred_element_type=jnp.float32)
            qb = (q * SCALE).astype(jnp.bfloat16)
            for g in range(HKV):
                Qs[b, g] = jnp.concatenate(
                    [qb[:, (g * GRP + j) * DH:(g * GRP + j + 1) * DH]
                     for j in range(GRP)], axis=0)

        st = {}
        ones_col = jnp.ones((SKV_LOC, 1), jnp.bfloat16)

        def process(b, g, Kc, Vc):
            s = lax.dot_general(
                Qs[b, g], Kc, (((1,), (1,)), ((), ())),
                preferred_element_type=jnp.float32)
            p = jnp.exp(s).astype(jnp.bfloat16)
            v_aug = jnp.concatenate([Vc, ones_col], axis=1)
            r = jnp.dot(p, v_aug,
                        preferred_element_type=jnp.float32)
            st[b, g] = r if (b, g) not in st else st[b, g] + r

        for b in range(B):
            for g in range(HKV):
                process(b, g,
                        kv_ref[b, :, g * DH:(g + 1) * DH],
                        kv_ref[b, :, KV_W + g * DH:KV_W + (g + 1) * DH])

        for d_rel in (1, 3, 2):
            origin = (my + d_rel) % N_DEV
            wr = pltpu.make_async_remote_copy(
                src_ref=kv_full.at[0],
                dst_ref=kv_full.at[origin],
                send_sem=send_sems.at[0],
                recv_sem=recv_sems.at[origin],
                device_id=(my,),
                device_id_type=pl.DeviceIdType.MESH,
            )
            wr.wait_recv()
            for b in range(B):
                for g in range(HKV):
                    process(b, g,
                            kv_full[origin, b, :, g * DH:(g + 1) * DH],
                            kv_full[origin, b, :,
                                    KV_W + g * DH:KV_W + (g + 1) * DH])

        for b in range(B):
            blocks = []
            for h in range(HQ):
                g, j = divmod(h, GRP)
                r = st[b, g]
                o = (r[:, 0:DH] / r[:, DH:DH + 1]).astype(jnp.bfloat16)
                blocks.append(o[j * SQ:(j + 1) * SQ, :])
            att = jnp.concatenate(blocks, axis=1)
            out_ref[b, :, :] = jnp.dot(att, wo_ref[...],
                                       preferred_element_type=jnp.float32)

        for rd in sends:
            rd.wait_send()

    return pl.pallas_call(
        body,
        out_shape=jax.ShapeDtypeStruct((B, SQ, D), jnp.float32),
        in_specs=[pl.BlockSpec(memory_space=pltpu.VMEM)] * 4,
        out_specs=pl.BlockSpec(memory_space=pltpu.VMEM),
        scratch_shapes=[
            pltpu.VMEM((N_DEV, B, SKV_LOC, 2 * KV_W), jnp.bfloat16),
            pltpu.SemaphoreType.DMA((N_DEV - 1,)),
            pltpu.SemaphoreType.DMA((N_DEV,)),
        ],
        compiler_params=pltpu.CompilerParams(collective_id=0),
    )(xb, wqb, wob, kv)
